# Initial kernel scaffold; baseline (speedup 1.0000x reference)
import jax
import jax.numpy as jnp
from jax.experimental import pallas as pl


def kernel(x, edge_index, node_type, edge_type, W, b):
    raise NotImplementedError("write your pallas kernel here")


# speedup vs baseline: 17.9236x; 17.9236x over previous
import functools

import jax
import jax.numpy as jnp
from jax import lax
from jax.experimental import pallas as pl
from jax.experimental.pallas import tpu as pltpu
from jax.experimental.pallas import tpu_sc as plsc

N_NODES = 10000
D = 128
DH = D // 2
NC, NS = 2, 16
NW = NC * NS
CHUNK = 128
NCHUNK_DEG = 80
NCHUNK_SC = 160
E_PAD = NW * NCHUNK_DEG * CHUNK
EROWS = E_PAD // CHUNK
ROWS_PAD = 10112
SR = ROWS_PAD // NS

_mesh = plsc.VectorSubcoreMesh(
    core_axis_name="c", subcore_axis_name="s", num_cores=NC, num_subcores=NS
)


@functools.partial(
    pl.kernel,
    out_type=jax.ShapeDtypeStruct((NC, ROWS_PAD, 16), jnp.float32),
    mesh=_mesh,
    scratch_types=[
        pltpu.VMEM((NCHUNK_DEG, CHUNK), jnp.int32),
        pltpu.VMEM((CHUNK, 16), jnp.float32),
        pltpu.VMEM_SHARED((ROWS_PAD, 16), jnp.float32),
    ],
    compiler_params=pltpu.CompilerParams(use_tc_tiling_on_sc=False),
)
def _sc_degree(dst_hbm, ones_hbm, zeros_hbm, cnt_hbm, dst_v, ones_v, deg_sh):
    cid = lax.axis_index("c")
    sid = lax.axis_index("s")
    wid = sid * NC + cid
    pltpu.sync_copy(zeros_hbm.at[pl.ds(sid * SR, SR)], deg_sh.at[pl.ds(sid * SR, SR)])
    pltpu.sync_copy(dst_hbm.at[pl.ds(wid * NCHUNK_DEG, NCHUNK_DEG)], dst_v)
    pltpu.sync_copy(ones_hbm, ones_v)
    plsc.subcore_barrier()

    def body(j, carry):
        pltpu.sync_copy(ones_v, deg_sh.at[dst_v.at[j]], add=True)
        return carry

    lax.fori_loop(0, NCHUNK_DEG, body, 0)
    plsc.subcore_barrier()
    pltpu.sync_copy(
        deg_sh.at[pl.ds(sid * SR, SR)], cnt_hbm.at[cid, pl.ds(sid * SR, SR)]
    )


@functools.partial(
    pl.kernel,
    out_type=jax.ShapeDtypeStruct((NC, ROWS_PAD, DH), jnp.float32),
    mesh=_mesh,
    scratch_types=[
        pltpu.VMEM((NCHUNK_SC, CHUNK), jnp.int32),
        pltpu.VMEM((NCHUNK_SC, CHUNK), jnp.int32),
        pltpu.VMEM((CHUNK, DH), jnp.float32),
        pltpu.VMEM((CHUNK, DH), jnp.float32),
        pltpu.SemaphoreType.DMA,
        pltpu.SemaphoreType.DMA,
        pltpu.VMEM_SHARED((ROWS_PAD, DH), jnp.float32),
    ],
    compiler_params=pltpu.CompilerParams(use_tc_tiling_on_sc=False),
)
def _sc_scatter(g_hbm, srcoff_hbm, dst_hbm, zeros_hbm, out_hbm,
                src_v, dst_v, buf0, buf1, sem0, sem1, acc_sh):
    cid = lax.axis_index("c")
    sid = lax.axis_index("s")
    pltpu.sync_copy(zeros_hbm.at[pl.ds(sid * SR, SR)], acc_sh.at[pl.ds(sid * SR, SR)])
    pltpu.sync_copy(
        srcoff_hbm.at[cid, pl.ds(sid * NCHUNK_SC, NCHUNK_SC)], src_v
    )
    pltpu.sync_copy(dst_hbm.at[pl.ds(sid * NCHUNK_SC, NCHUNK_SC)], dst_v)
    plsc.subcore_barrier()

    pltpu.async_copy(g_hbm.at[src_v.at[0]], buf0, sem0)
    pltpu.async_copy(g_hbm.at[src_v.at[1]], buf1, sem1)

    def pair(i, carry):
        j = 2 * i
        pltpu.make_async_copy(g_hbm.at[src_v.at[j]], buf0, sem0).wait()
        pltpu.sync_copy(buf0, acc_sh.at[dst_v.at[j]], add=True)
        pltpu.async_copy(g_hbm.at[src_v.at[j + 2]], buf0, sem0)
        pltpu.make_async_copy(g_hbm.at[src_v.at[j + 1]], buf1, sem1).wait()
        pltpu.sync_copy(buf1, acc_sh.at[dst_v.at[j + 1]], add=True)
        pltpu.async_copy(g_hbm.at[src_v.at[j + 3]], buf1, sem1)
        return carry

    lax.fori_loop(0, NCHUNK_SC // 2 - 1, pair, 0)
    pltpu.make_async_copy(g_hbm.at[src_v.at[NCHUNK_SC - 2]], buf0, sem0).wait()
    pltpu.sync_copy(buf0, acc_sh.at[dst_v.at[NCHUNK_SC - 2]], add=True)
    pltpu.make_async_copy(g_hbm.at[src_v.at[NCHUNK_SC - 1]], buf1, sem1).wait()
    pltpu.sync_copy(buf1, acc_sh.at[dst_v.at[NCHUNK_SC - 1]], add=True)
    plsc.subcore_barrier()
    pltpu.sync_copy(
        acc_sh.at[pl.ds(sid * SR, SR)], out_hbm.at[cid, pl.ds(sid * SR, SR)]
    )


BR = 1000


def _tc_transform_body(x_ref, w_ref, c0_ref, c1_ref, g_ref, dinv_ref):
    dinv16 = lax.rsqrt(1.0 + c0_ref[...] + c1_ref[...])
    h = jnp.dot(x_ref[...], w_ref[0], preferred_element_type=jnp.float32)
    g_ref[0] = h * dinv16[:, 0:1]
    dinv_ref[...] = dinv16


_tc_transform = pl.pallas_call(
    _tc_transform_body,
    grid=(NC, N_NODES // BR),
    in_specs=[
        pl.BlockSpec((BR, D), lambda c, i: (i, 0)),
        pl.BlockSpec((1, D, DH), lambda c, i: (c, 0, 0)),
        pl.BlockSpec((BR, 16), lambda c, i: (i, 0)),
        pl.BlockSpec((BR, 16), lambda c, i: (i, 0)),
    ],
    out_specs=[
        pl.BlockSpec((1, BR, DH), lambda c, i: (c, i, 0)),
        pl.BlockSpec((BR, 16), lambda c, i: (i, 0)),
    ],
    out_shape=[
        jax.ShapeDtypeStruct((NC, N_NODES, DH), jnp.float32),
        jax.ShapeDtypeStruct((N_NODES, 16), jnp.float32),
    ],
)


def _tc_final_body(a0_ref, a1_ref, g0_ref, g1_ref, dinv_ref, b_ref, out_ref):
    dinv = dinv_ref[:, 0:1]
    left = dinv * (a0_ref[0] + g0_ref[0])
    right = dinv * (a1_ref[0] + g1_ref[0])
    out_ref[...] = jnp.concatenate([left, right], axis=1) + b_ref[...]


_tc_final = pl.pallas_call(
    _tc_final_body,
    grid=(N_NODES // BR,),
    in_specs=[
        pl.BlockSpec((1, BR, DH), lambda i: (0, i, 0)),
        pl.BlockSpec((1, BR, DH), lambda i: (1, i, 0)),
        pl.BlockSpec((1, BR, DH), lambda i: (0, i, 0)),
        pl.BlockSpec((1, BR, DH), lambda i: (1, i, 0)),
        pl.BlockSpec((BR, 16), lambda i: (i, 0)),
        pl.BlockSpec((1, D), lambda i: (0, 0)),
    ],
    out_specs=pl.BlockSpec((BR, D), lambda i: (i, 0)),
    out_shape=jax.ShapeDtypeStruct((N_NODES, D), jnp.float32),
)


def kernel(x, edge_index, node_type, edge_type, W, b):
    del node_type, edge_type
    ei = edge_index.astype(jnp.int32)
    pad = E_PAD - ei.shape[1]
    src = jnp.concatenate([ei[0], jnp.zeros((pad,), jnp.int32)])
    dst = jnp.concatenate([ei[1], jnp.full((pad,), N_NODES, jnp.int32)])
    src2d = src.reshape(EROWS, CHUNK)
    dst2d = dst.reshape(EROWS, CHUNK)
    srcoff = jnp.stack([src2d, src2d + N_NODES])
    ones16 = jnp.ones((CHUNK, 16), jnp.float32)
    zeros16 = jnp.zeros((ROWS_PAD, 16), jnp.float32)
    zeros64 = jnp.zeros((ROWS_PAD, DH), jnp.float32)

    Ws = jnp.stack([W[:, :DH], W[:, DH:]])
    cnt = _sc_degree(dst2d, ones16, zeros16)
    g, dinv16 = _tc_transform(x, Ws, cnt[0], cnt[1])
    g2 = g.reshape(NC * N_NODES, DH)
    acc = _sc_scatter(g2, srcoff, dst2d, zeros64)
    out = _tc_final(acc, acc, g, g, dinv16, b.reshape(1, D))
    return out
